# baseline (speedup 1.0000x reference)
import functools

import jax
import jax.numpy as jnp
from jax import lax
from jax.experimental import pallas as pl
from jax.experimental.pallas import tpu as pltpu
from jax.experimental.pallas import tpu_sc as plsc

L = 16


def _tc_body(seq_ref, b_ref, a_ref, v_ref, w_ref,
             a_sls_ref, v_sls_ref, av_sls_ref, key_ref, *, b0):
    i = pl.program_id(0)
    half = a_ref.shape[2]

    w = w_ref[...]
    bb = b_ref[0]
    s = seq_ref[b0 + i]
    for h in range(2):
        a2 = a_ref[0, h]
        v2 = v_ref[0, h]
        la = jnp.dot(a2, w, preferred_element_type=jnp.float32) + bb
        lv = jnp.dot(v2, w, preferred_element_type=jnp.float32) + bb
        av = la + lv
        a_sls_ref[0, h] = jax.nn.sigmoid(la)
        v_sls_ref[0, h] = jax.nn.sigmoid(lv)
        av_sls_ref[0, h] = av
        pos = lax.broadcasted_iota(jnp.int32, (half, 1), 0) + h * half
        bits = lax.bitcast_convert_type(av, jnp.uint32)
        ukey = jnp.where(bits >= jnp.uint32(0x80000000), ~bits,
                         bits | jnp.uint32(0x80000000))
        key_ref[0, h] = jnp.where(pos < s, ukey, jnp.uint32(0))


def _tc_call(a4, v4, seq_len, W, b, b0, nb):
    _, _, half, D = a4.shape
    in_spec = pl.BlockSpec((1, 2, half, D), lambda i: (b0 + i, 0, 0, 0))
    out_spec = pl.BlockSpec((1, 2, half, 1), lambda i: (i, 0, 0, 0))
    return pl.pallas_call(
        functools.partial(_tc_body, b0=b0),
        grid=(nb,),
        in_specs=[
            pl.BlockSpec(memory_space=pltpu.SMEM),
            pl.BlockSpec(memory_space=pltpu.SMEM),
            in_spec, in_spec,
            pl.BlockSpec((D, 1), lambda i: (0, 0)),
        ],
        out_specs=[out_spec, out_spec, out_spec, out_spec],
        out_shape=[
            jax.ShapeDtypeStruct((nb, 2, half, 1), jnp.float32),
            jax.ShapeDtypeStruct((nb, 2, half, 1), jnp.float32),
            jax.ShapeDtypeStruct((nb, 2, half, 1), jnp.float32),
            jax.ShapeDtypeStruct((nb, 2, half, 1), jnp.uint32),
        ],
        compiler_params=pltpu.CompilerParams(
            dimension_semantics=("parallel",)),
    )(seq_len, b, a4, v4, W)


def _make_sc_topk(nb, T, b0):
    NV = T // L
    mesh = plsc.VectorSubcoreMesh(core_axis_name="c", subcore_axis_name="s")

    @functools.partial(
        pl.kernel,
        mesh=mesh,
        out_type=jax.ShapeDtypeStruct((nb, L), jnp.float32),
        compiler_params=pltpu.CompilerParams(needs_layout_passes=False),
        scratch_types=[
            pltpu.VMEM((T,), jnp.uint32),
            pltpu.VMEM((L,), jnp.int32),
            pltpu.VMEM((256,), jnp.int32),
            pltpu.VMEM((L,), jnp.float32),
        ],
    )
    def sc_topk(keys_hbm, seq_hbm, out_hbm, row_v, seq_v, hist_v, out_v):
        c = lax.axis_index("c")
        sub = lax.axis_index("s")
        wid = sub * 2 + c

        @pl.when(wid < nb)
        def _():
            pltpu.sync_copy(keys_hbm.at[wid], row_v)
            pltpu.sync_copy(seq_hbm, seq_v)
            iota = lax.iota(jnp.int32, L)
            s = jnp.sum(jnp.where(iota == b0 + wid, seq_v[...], jnp.int32(0)))
            k = s // 16 + 1

            prefix = jnp.uint32(0)
            r = k
            for shift, himask in ((24, 0x00000000), (16, 0xFF000000),
                                  (8, 0xFFFF0000), (0, 0xFFFFFF00)):
                def zero_body(vv, carry):
                    hist_v[pl.ds(vv * L, L)] = jnp.zeros((L,), jnp.int32)
                    return carry
                lax.fori_loop(0, 256 // L, zero_body, 0)

                hm = jnp.uint32(himask)
                pfx = prefix

                def hist_body(ii, carry):
                    u = row_v[pl.ds(ii * L, L)]
                    match = (u & hm) == pfx
                    byte = ((u >> shift) & jnp.uint32(0xFF)).astype(jnp.int32)
                    add = jnp.where(match, jnp.int32(1), jnp.int32(0))
                    plsc.addupdate_scatter(hist_v, [byte], add)
                    return carry
                lax.fori_loop(0, NV, hist_body, 0)

                def scan_body(t, sc):
                    cum, b, sb1, found = sc
                    v = 15 - t
                    h = hist_v[pl.ds(v * L, L)]
                    ssum = lax.rev(jnp.cumsum(lax.rev(h, (0,))), (0,))
                    Wv = ssum + cum
                    mask = Wv >= r
                    ntrue = jnp.max(plsc.all_reduce_population_count(mask))
                    found_here = ntrue > 0
                    b_here = v * L + ntrue - 1
                    w_at = jnp.sum(jnp.where(iota == ntrue, Wv, jnp.int32(0)))
                    sb1_here = jnp.where(ntrue == L, cum, w_at)
                    take = jnp.logical_and(found_here, jnp.logical_not(found))
                    b = jnp.where(take, b_here, b)
                    sb1 = jnp.where(take, sb1_here, sb1)
                    found = jnp.logical_or(found, found_here)
                    cum = jnp.max(Wv)
                    return (cum, b, sb1, found)

                _, b, sb1, _ = lax.fori_loop(
                    0, 256 // L, scan_body,
                    (jnp.int32(0), jnp.int32(0), jnp.int32(0), jnp.bool_(False)))
                prefix = prefix | (b.astype(jnp.uint32) << shift)
                r = r - sb1

            pfx_vec = jnp.full((L,), prefix, jnp.uint32)

            def sum_body(ii, acc):
                u = row_v[pl.ds(ii * L, L)]
                gt = u > pfx_vec
                bits = jnp.where(u >= jnp.uint32(0x80000000),
                                 u ^ jnp.uint32(0x80000000), ~u)
                x = lax.bitcast_convert_type(bits, jnp.float32)
                return acc + jnp.where(gt, x, jnp.float32(0.0))

            acc = lax.fori_loop(0, NV, sum_body, jnp.zeros((L,), jnp.float32))
            total = jnp.sum(acc)

            tbits = jnp.where(pfx_vec >= jnp.uint32(0x80000000),
                              pfx_vec ^ jnp.uint32(0x80000000), ~pfx_vec)
            thresh = lax.bitcast_convert_type(tbits, jnp.float32)
            z = (total + r.astype(jnp.float32) * thresh) / k.astype(jnp.float32)
            out_v[...] = 1.0 / (1.0 + jnp.exp(-z))
            pltpu.sync_copy(out_v, out_hbm.at[wid])

    return sc_topk


def kernel(a_out, v_out, seq_len, W, b):
    Bn, T, D = a_out.shape
    half = T // 2
    a4 = a_out.reshape(Bn, 2, half, D)
    v4 = v_out.reshape(Bn, 2, half, D)
    nb = Bn // 2

    parts = []
    for hb in range(2):
        b0 = hb * nb
        a_sls, v_sls, av_sls, keys = _tc_call(a4, v4, seq_len, W, b, b0, nb)
        mil = _make_sc_topk(nb, T, b0)(keys.reshape(nb, T), seq_len)
        parts.append((mil[:, 0], a_sls.reshape(nb, T, 1),
                      v_sls.reshape(nb, T, 1), av_sls.reshape(nb, T, 1)))

    (m0, a0, v0, av0), (m1, a1, v1, av1) = parts
    return (jnp.concatenate([m0, m1]),
            jnp.concatenate([a0, a1]),
            jnp.concatenate([v0, v1]),
            jnp.concatenate([av0, av1]))


# speedup vs baseline: 1.0738x; 1.0738x over previous
import functools

import jax
import jax.numpy as jnp
from jax import lax
from jax.experimental import pallas as pl
from jax.experimental.pallas import tpu as pltpu
from jax.experimental.pallas import tpu_sc as plsc

L = 16
NCH = 8


def _tc_body(seq_ref, b_ref, a_hbm, v_hbm, w_ref,
             a_sls_ref, v_sls_ref, av_sls_ref, key_ref,
             a_buf, v_buf, sems):
    i = pl.program_id(0)
    nb = pl.num_programs(0)
    T = a_buf.shape[1]
    ch = T // NCH

    def issue(bag, slot):
        for inp, (hbm, buf) in enumerate(((a_hbm, a_buf), (v_hbm, v_buf))):
            for c in range(NCH):
                pltpu.make_async_copy(
                    hbm.at[bag, pl.ds(c * ch, ch)],
                    buf.at[slot, pl.ds(c * ch, ch)],
                    sems.at[slot, inp, c],
                ).start()

    def drain(bag, slot):
        for inp, (hbm, buf) in enumerate(((a_hbm, a_buf), (v_hbm, v_buf))):
            for c in range(NCH):
                pltpu.make_async_copy(
                    hbm.at[bag, pl.ds(c * ch, ch)],
                    buf.at[slot, pl.ds(c * ch, ch)],
                    sems.at[slot, inp, c],
                ).wait()

    slot = lax.rem(i, 2)
    nxt = lax.rem(i + 1, 2)

    @pl.when(i == 0)
    def _():
        issue(0, 0)

    @pl.when(i + 1 < nb)
    def _():
        issue(i + 1, nxt)

    drain(i, slot)

    w = w_ref[...]
    bb = b_ref[0]
    s = seq_ref[i]
    a2 = a_buf[slot]
    v2 = v_buf[slot]
    la = jnp.dot(a2, w, preferred_element_type=jnp.float32) + bb
    lv = jnp.dot(v2, w, preferred_element_type=jnp.float32) + bb
    av = la + lv
    a_sls_ref[0] = jax.nn.sigmoid(la)
    v_sls_ref[0] = jax.nn.sigmoid(lv)
    av_sls_ref[0] = av
    pos = lax.broadcasted_iota(jnp.int32, (T, 1), 0)
    bits = lax.bitcast_convert_type(av, jnp.uint32)
    ukey = jnp.where(bits >= jnp.uint32(0x80000000), ~bits,
                     bits | jnp.uint32(0x80000000))
    key_ref[0] = jnp.where(pos < s, ukey, jnp.uint32(0))


def _tc_call(a_out, v_out, seq_len, W, b):
    Bn, T, D = a_out.shape
    out_spec = pl.BlockSpec((1, T, 1), lambda i: (i, 0, 0))
    return pl.pallas_call(
        _tc_body,
        grid=(Bn,),
        in_specs=[
            pl.BlockSpec(memory_space=pltpu.SMEM),
            pl.BlockSpec(memory_space=pltpu.SMEM),
            pl.BlockSpec(memory_space=pl.ANY),
            pl.BlockSpec(memory_space=pl.ANY),
            pl.BlockSpec((D, 1), lambda i: (0, 0)),
        ],
        out_specs=[out_spec, out_spec, out_spec, out_spec],
        out_shape=[
            jax.ShapeDtypeStruct((Bn, T, 1), jnp.float32),
            jax.ShapeDtypeStruct((Bn, T, 1), jnp.float32),
            jax.ShapeDtypeStruct((Bn, T, 1), jnp.float32),
            jax.ShapeDtypeStruct((Bn, T, 1), jnp.uint32),
        ],
        scratch_shapes=[
            pltpu.VMEM((2, T, D), jnp.float32),
            pltpu.VMEM((2, T, D), jnp.float32),
            pltpu.SemaphoreType.DMA((2, 2, NCH)),
        ],
        compiler_params=pltpu.CompilerParams(
            dimension_semantics=("arbitrary",)),
    )(seq_len, b, a_out, v_out, W)


def _make_sc_topk(Bn, T):
    NV = T // L
    mesh = plsc.VectorSubcoreMesh(core_axis_name="c", subcore_axis_name="s")

    @functools.partial(
        pl.kernel,
        mesh=mesh,
        out_type=jax.ShapeDtypeStruct((Bn, L), jnp.float32),
        compiler_params=pltpu.CompilerParams(needs_layout_passes=False),
        scratch_types=[
            pltpu.VMEM((T,), jnp.uint32),
            pltpu.VMEM((L,), jnp.int32),
            pltpu.VMEM((256,), jnp.int32),
            pltpu.VMEM((L,), jnp.float32),
        ],
    )
    def sc_topk(keys_hbm, seq_hbm, out_hbm, row_v, seq_v, hist_v, out_v):
        c = lax.axis_index("c")
        sub = lax.axis_index("s")
        wid = sub * 2 + c

        @pl.when(wid < Bn)
        def _():
            pltpu.sync_copy(keys_hbm.at[wid], row_v)
            pltpu.sync_copy(seq_hbm, seq_v)
            iota = lax.iota(jnp.int32, L)
            s = jnp.sum(jnp.where(iota == wid, seq_v[...], jnp.int32(0)))
            k = s // 16 + 1

            prefix = jnp.uint32(0)
            r = k
            for shift, himask in ((24, 0x00000000), (16, 0xFF000000),
                                  (8, 0xFFFF0000), (0, 0xFFFFFF00)):
                def zero_body(vv, carry):
                    hist_v[pl.ds(vv * L, L)] = jnp.zeros((L,), jnp.int32)
                    return carry
                lax.fori_loop(0, 256 // L, zero_body, 0)

                hm = jnp.uint32(himask)
                pfx = prefix

                def hist_body(ii, carry):
                    u = row_v[pl.ds(ii * L, L)]
                    match = (u & hm) == pfx
                    byte = ((u >> shift) & jnp.uint32(0xFF)).astype(jnp.int32)
                    add = jnp.where(match, jnp.int32(1), jnp.int32(0))
                    plsc.addupdate_scatter(hist_v, [byte], add)
                    return carry
                lax.fori_loop(0, NV, hist_body, 0)

                def scan_body(t, sc):
                    cum, b, sb1, found = sc
                    v = 15 - t
                    h = hist_v[pl.ds(v * L, L)]
                    ssum = lax.rev(jnp.cumsum(lax.rev(h, (0,))), (0,))
                    Wv = ssum + cum
                    mask = Wv >= r
                    ntrue = jnp.max(plsc.all_reduce_population_count(mask))
                    found_here = ntrue > 0
                    b_here = v * L + ntrue - 1
                    w_at = jnp.sum(jnp.where(iota == ntrue, Wv, jnp.int32(0)))
                    sb1_here = jnp.where(ntrue == L, cum, w_at)
                    take = jnp.logical_and(found_here, jnp.logical_not(found))
                    b = jnp.where(take, b_here, b)
                    sb1 = jnp.where(take, sb1_here, sb1)
                    found = jnp.logical_or(found, found_here)
                    cum = jnp.max(Wv)
                    return (cum, b, sb1, found)

                _, b, sb1, _ = lax.fori_loop(
                    0, 256 // L, scan_body,
                    (jnp.int32(0), jnp.int32(0), jnp.int32(0), jnp.bool_(False)))
                prefix = prefix | (b.astype(jnp.uint32) << shift)
                r = r - sb1

            pfx_vec = jnp.full((L,), prefix, jnp.uint32)

            def sum_body(ii, acc):
                u = row_v[pl.ds(ii * L, L)]
                gt = u > pfx_vec
                bits = jnp.where(u >= jnp.uint32(0x80000000),
                                 u ^ jnp.uint32(0x80000000), ~u)
                x = lax.bitcast_convert_type(bits, jnp.float32)
                return acc + jnp.where(gt, x, jnp.float32(0.0))

            acc = lax.fori_loop(0, NV, sum_body, jnp.zeros((L,), jnp.float32))
            total = jnp.sum(acc)

            tbits = jnp.where(pfx_vec >= jnp.uint32(0x80000000),
                              pfx_vec ^ jnp.uint32(0x80000000), ~pfx_vec)
            thresh = lax.bitcast_convert_type(tbits, jnp.float32)
            z = (total + r.astype(jnp.float32) * thresh) / k.astype(jnp.float32)
            out_v[...] = 1.0 / (1.0 + jnp.exp(-z))
            pltpu.sync_copy(out_v, out_hbm.at[wid])

    return sc_topk


def kernel(a_out, v_out, seq_len, W, b):
    Bn, T, D = a_out.shape
    a_sls, v_sls, av_sls, keys = _tc_call(a_out, v_out, seq_len, W, b)
    mil_mat = _make_sc_topk(Bn, T)(keys.reshape(Bn, T), seq_len)
    return (mil_mat[:, 0], a_sls, v_sls, av_sls)


# speedup vs baseline: 1.1121x; 1.0357x over previous
import functools

import jax
import jax.numpy as jnp
from jax import lax
from jax.experimental import pallas as pl
from jax.experimental.pallas import tpu as pltpu
from jax.experimental.pallas import tpu_sc as plsc

L = 16
UNR = 8


def _tc_body(seq_ref, b_ref, a0_ref, a1_ref, v0_ref, v1_ref, w_ref,
             a_sls_ref, v_sls_ref, av_sls_ref, key_ref):
    i = pl.program_id(0)
    half = a0_ref.shape[2]

    w = w_ref[...]
    bb = b_ref[0]
    s = seq_ref[i]
    for h in range(2):
        a2 = (a0_ref, a1_ref)[h][0, 0]
        v2 = (v0_ref, v1_ref)[h][0, 0]
        la = jnp.dot(a2, w, preferred_element_type=jnp.float32) + bb
        lv = jnp.dot(v2, w, preferred_element_type=jnp.float32) + bb
        av = la + lv
        a_sls_ref[0, h] = jax.nn.sigmoid(la)
        v_sls_ref[0, h] = jax.nn.sigmoid(lv)
        av_sls_ref[0, h] = av
        pos = lax.broadcasted_iota(jnp.int32, (half, 1), 0) + h * half
        bits = lax.bitcast_convert_type(av, jnp.uint32)
        ukey = jnp.where(bits >= jnp.uint32(0x80000000), ~bits,
                         bits | jnp.uint32(0x80000000))
        key_ref[0, h] = jnp.where(pos < s, ukey, jnp.uint32(0))


def _tc_call(a_out, v_out, seq_len, W, b):
    Bn, T, D = a_out.shape
    half = T // 2
    a4 = a_out.reshape(Bn, 2, half, D)
    v4 = v_out.reshape(Bn, 2, half, D)
    half_spec = lambda h: pl.BlockSpec((1, 1, half, D),
                                       lambda i, h=h: (i, h, 0, 0))
    out_spec = pl.BlockSpec((1, 2, half, 1), lambda i: (i, 0, 0, 0))
    outs = pl.pallas_call(
        _tc_body,
        grid=(Bn,),
        in_specs=[
            pl.BlockSpec(memory_space=pltpu.SMEM),
            pl.BlockSpec(memory_space=pltpu.SMEM),
            half_spec(0), half_spec(1),
            half_spec(0), half_spec(1),
            pl.BlockSpec((D, 1), lambda i: (0, 0)),
        ],
        out_specs=[out_spec, out_spec, out_spec, out_spec],
        out_shape=[
            jax.ShapeDtypeStruct((Bn, 2, half, 1), jnp.float32),
            jax.ShapeDtypeStruct((Bn, 2, half, 1), jnp.float32),
            jax.ShapeDtypeStruct((Bn, 2, half, 1), jnp.float32),
            jax.ShapeDtypeStruct((Bn, 2, half, 1), jnp.uint32),
        ],
        compiler_params=pltpu.CompilerParams(
            dimension_semantics=("parallel",)),
    )(seq_len, b, a4, a4, v4, v4, W)
    return [o.reshape(Bn, T, 1) for o in outs]


def _make_sc_topk(Bn, T):
    NV = T // L
    mesh = plsc.VectorSubcoreMesh(core_axis_name="c", subcore_axis_name="s")

    @functools.partial(
        pl.kernel,
        mesh=mesh,
        out_type=jax.ShapeDtypeStruct((Bn, L), jnp.float32),
        compiler_params=pltpu.CompilerParams(needs_layout_passes=False),
        scratch_types=[
            pltpu.VMEM((T,), jnp.uint32),
            pltpu.VMEM((L,), jnp.int32),
            pltpu.VMEM((256,), jnp.int32),
            pltpu.VMEM((L,), jnp.float32),
        ],
    )
    def sc_topk(keys_hbm, seq_hbm, out_hbm, row_v, seq_v, hist_v, out_v):
        c = lax.axis_index("c")
        sub = lax.axis_index("s")
        wid = sub * 2 + c

        @pl.when(wid < Bn)
        def _():
            pltpu.sync_copy(keys_hbm.at[wid], row_v)
            pltpu.sync_copy(seq_hbm, seq_v)
            iota = lax.iota(jnp.int32, L)
            s = jnp.sum(jnp.where(iota == wid, seq_v[...], jnp.int32(0)))
            k = s // 16 + 1

            prefix = jnp.uint32(0)
            r = k
            for shift, himask in ((24, 0x00000000), (16, 0xFF000000),
                                  (8, 0xFFFF0000), (0, 0xFFFFFF00)):
                for vv in range(256 // L):
                    hist_v[pl.ds(vv * L, L)] = jnp.zeros((L,), jnp.int32)

                hm = jnp.uint32(himask)
                pfx = prefix

                def hist_body(ii, carry):
                    for q in range(UNR):
                        u = row_v[pl.ds((ii * UNR + q) * L, L)]
                        match = (u & hm) == pfx
                        byte = ((u >> shift) & jnp.uint32(0xFF)).astype(jnp.int32)
                        add = jnp.where(match, jnp.int32(1), jnp.int32(0))
                        plsc.addupdate_scatter(hist_v, [byte], add)
                    return carry
                lax.fori_loop(0, NV // UNR, hist_body, 0)

                def scan_body(t, sc):
                    cum, b, sb1, found = sc
                    v = 15 - t
                    h = hist_v[pl.ds(v * L, L)]
                    ssum = lax.rev(jnp.cumsum(lax.rev(h, (0,))), (0,))
                    Wv = ssum + cum
                    mask = Wv >= r
                    ntrue = jnp.max(plsc.all_reduce_population_count(mask))
                    found_here = ntrue > 0
                    b_here = v * L + ntrue - 1
                    w_at = jnp.sum(jnp.where(iota == ntrue, Wv, jnp.int32(0)))
                    sb1_here = jnp.where(ntrue == L, cum, w_at)
                    take = jnp.logical_and(found_here, jnp.logical_not(found))
                    b = jnp.where(take, b_here, b)
                    sb1 = jnp.where(take, sb1_here, sb1)
                    found = jnp.logical_or(found, found_here)
                    cum = jnp.max(Wv)
                    return (cum, b, sb1, found)

                _, b, sb1, _ = lax.fori_loop(
                    0, 256 // L, scan_body,
                    (jnp.int32(0), jnp.int32(0), jnp.int32(0), jnp.bool_(False)))
                prefix = prefix | (b.astype(jnp.uint32) << shift)
                r = r - sb1

            pfx_vec = jnp.full((L,), prefix, jnp.uint32)

            def sum_body(ii, acc):
                for q in range(UNR):
                    u = row_v[pl.ds((ii * UNR + q) * L, L)]
                    gt = u > pfx_vec
                    bits = jnp.where(u >= jnp.uint32(0x80000000),
                                     u ^ jnp.uint32(0x80000000), ~u)
                    x = lax.bitcast_convert_type(bits, jnp.float32)
                    acc = acc + jnp.where(gt, x, jnp.float32(0.0))
                return acc
            acc = lax.fori_loop(0, NV // UNR, sum_body,
                                jnp.zeros((L,), jnp.float32))
            total = jnp.sum(acc)

            tbits = jnp.where(pfx_vec >= jnp.uint32(0x80000000),
                              pfx_vec ^ jnp.uint32(0x80000000), ~pfx_vec)
            thresh = lax.bitcast_convert_type(tbits, jnp.float32)
            z = (total + r.astype(jnp.float32) * thresh) / k.astype(jnp.float32)
            out_v[...] = 1.0 / (1.0 + jnp.exp(-z))
            pltpu.sync_copy(out_v, out_hbm.at[wid])

    return sc_topk


def kernel(a_out, v_out, seq_len, W, b):
    Bn, T, D = a_out.shape
    a_sls, v_sls, av_sls, keys = _tc_call(a_out, v_out, seq_len, W, b)
    mil_mat = _make_sc_topk(Bn, T)(keys.reshape(Bn, T), seq_len)
    return (mil_mat[:, 0], a_sls, v_sls, av_sls)


# speedup vs baseline: 1.1270x; 1.0134x over previous
import functools

import jax
import jax.numpy as jnp
from jax import lax
from jax.experimental import pallas as pl
from jax.experimental.pallas import tpu as pltpu
from jax.experimental.pallas import tpu_sc as plsc

L = 16
UNR = 8


def _tc_body(seq_ref, b_ref, a0_ref, a1_ref, v0_ref, v1_ref, w_ref,
             a_sls_ref, v_sls_ref, av_sls_ref, key_ref):
    i = pl.program_id(0)
    half = a0_ref.shape[2]

    w = w_ref[...]
    bb = b_ref[0]
    s = seq_ref[i]
    for h in range(2):
        a2 = (a0_ref, a1_ref)[h][0, 0]
        v2 = (v0_ref, v1_ref)[h][0, 0]
        la = jnp.dot(a2, w, preferred_element_type=jnp.float32) + bb
        lv = jnp.dot(v2, w, preferred_element_type=jnp.float32) + bb
        av = la + lv
        a_sls_ref[0, h] = jax.nn.sigmoid(la)
        v_sls_ref[0, h] = jax.nn.sigmoid(lv)
        av_sls_ref[0, h] = av
        pos = lax.broadcasted_iota(jnp.int32, (half, 1), 0) + h * half
        bits = lax.bitcast_convert_type(av, jnp.uint32)
        ukey = jnp.where(bits >= jnp.uint32(0x80000000), ~bits,
                         bits | jnp.uint32(0x80000000))
        key_ref[0, h] = jnp.where(pos < s, ukey, jnp.uint32(0))


def _tc_call(a_out, v_out, seq_len, W, b):
    Bn, T, D = a_out.shape
    half = T // 2
    a4 = a_out.reshape(Bn, 2, half, D)
    v4 = v_out.reshape(Bn, 2, half, D)
    half_spec = lambda h: pl.BlockSpec((1, 1, half, D),
                                       lambda i, h=h: (i, h, 0, 0))
    out_spec = pl.BlockSpec((1, 2, half, 1), lambda i: (i, 0, 0, 0))
    outs = pl.pallas_call(
        _tc_body,
        grid=(Bn,),
        in_specs=[
            pl.BlockSpec(memory_space=pltpu.SMEM),
            pl.BlockSpec(memory_space=pltpu.SMEM),
            half_spec(0), half_spec(1),
            half_spec(0), half_spec(1),
            pl.BlockSpec((D, 1), lambda i: (0, 0)),
        ],
        out_specs=[out_spec, out_spec, out_spec, out_spec],
        out_shape=[
            jax.ShapeDtypeStruct((Bn, 2, half, 1), jnp.float32),
            jax.ShapeDtypeStruct((Bn, 2, half, 1), jnp.float32),
            jax.ShapeDtypeStruct((Bn, 2, half, 1), jnp.float32),
            jax.ShapeDtypeStruct((Bn, 2, half, 1), jnp.uint32),
        ],
        compiler_params=pltpu.CompilerParams(
            dimension_semantics=("parallel",)),
    )(seq_len, b, a4, a4, v4, v4, W)
    return [o.reshape(Bn, T, 1) for o in outs]


def _make_sc_topk(Bn, T):
    NV = T // L
    mesh = plsc.VectorSubcoreMesh(core_axis_name="c", subcore_axis_name="s")

    @functools.partial(
        pl.kernel,
        mesh=mesh,
        out_type=jax.ShapeDtypeStruct((Bn,), jnp.float32),
        compiler_params=pltpu.CompilerParams(needs_layout_passes=False),
        scratch_types=[
            pltpu.VMEM((T,), jnp.uint32),
            pltpu.VMEM((L,), jnp.int32),
            pltpu.VMEM((256,), jnp.int32),
            pltpu.VMEM((L,), jnp.float32),
            pltpu.VMEM((L, L), jnp.float32),
            pltpu.VMEM_SHARED((L, L), jnp.float32),
        ],
    )
    def sc_topk(keys_hbm, seq_hbm, out_hbm, row_v, seq_v, hist_v, out_v,
                gat_v, shared):
        c = lax.axis_index("c")
        sub = lax.axis_index("s")
        wid = sub

        @pl.when(c == 0)
        def _():
            pltpu.sync_copy(keys_hbm.at[wid], row_v)
            pltpu.sync_copy(seq_hbm, seq_v)
            iota = lax.iota(jnp.int32, L)
            s = jnp.sum(jnp.where(iota == wid, seq_v[...], jnp.int32(0)))
            k = s // 16 + 1

            prefix = jnp.uint32(0)
            r = k
            for shift, himask in ((24, 0x00000000), (16, 0xFF000000),
                                  (8, 0xFFFF0000), (0, 0xFFFFFF00)):
                for vv in range(256 // L):
                    hist_v[pl.ds(vv * L, L)] = jnp.zeros((L,), jnp.int32)

                hm = jnp.uint32(himask)
                pfx = prefix

                def hist_body(ii, carry):
                    for q in range(UNR):
                        u = row_v[pl.ds((ii * UNR + q) * L, L)]
                        match = (u & hm) == pfx
                        byte = ((u >> shift) & jnp.uint32(0xFF)).astype(jnp.int32)
                        add = jnp.where(match, jnp.int32(1), jnp.int32(0))
                        plsc.addupdate_scatter(hist_v, [byte], add)
                    return carry
                lax.fori_loop(0, NV // UNR, hist_body, 0)

                def scan_body(t, sc):
                    cum, b, sb1, found = sc
                    v = 15 - t
                    h = hist_v[pl.ds(v * L, L)]
                    ssum = lax.rev(jnp.cumsum(lax.rev(h, (0,))), (0,))
                    Wv = ssum + cum
                    mask = Wv >= r
                    ntrue = jnp.max(plsc.all_reduce_population_count(mask))
                    found_here = ntrue > 0
                    b_here = v * L + ntrue - 1
                    w_at = jnp.sum(jnp.where(iota == ntrue, Wv, jnp.int32(0)))
                    sb1_here = jnp.where(ntrue == L, cum, w_at)
                    take = jnp.logical_and(found_here, jnp.logical_not(found))
                    b = jnp.where(take, b_here, b)
                    sb1 = jnp.where(take, sb1_here, sb1)
                    found = jnp.logical_or(found, found_here)
                    cum = jnp.max(Wv)
                    return (cum, b, sb1, found)

                _, b, sb1, _ = lax.fori_loop(
                    0, 256 // L, scan_body,
                    (jnp.int32(0), jnp.int32(0), jnp.int32(0), jnp.bool_(False)))
                prefix = prefix | (b.astype(jnp.uint32) << shift)
                r = r - sb1

            pfx_vec = jnp.full((L,), prefix, jnp.uint32)

            def sum_body(ii, acc):
                for q in range(UNR):
                    u = row_v[pl.ds((ii * UNR + q) * L, L)]
                    gt = u > pfx_vec
                    bits = jnp.where(u >= jnp.uint32(0x80000000),
                                     u ^ jnp.uint32(0x80000000), ~u)
                    x = lax.bitcast_convert_type(bits, jnp.float32)
                    acc = acc + jnp.where(gt, x, jnp.float32(0.0))
                return acc
            acc = lax.fori_loop(0, NV // UNR, sum_body,
                                jnp.zeros((L,), jnp.float32))
            total = jnp.sum(acc)

            tbits = jnp.where(pfx_vec >= jnp.uint32(0x80000000),
                              pfx_vec ^ jnp.uint32(0x80000000), ~pfx_vec)
            thresh = lax.bitcast_convert_type(tbits, jnp.float32)
            z = (total + r.astype(jnp.float32) * thresh) / k.astype(jnp.float32)
            out_v[...] = 1.0 / (1.0 + jnp.exp(-z))
            pltpu.sync_copy(out_v, shared.at[wid])
            plsc.subcore_barrier()

            @pl.when(sub == 0)
            def _():
                pltpu.sync_copy(shared, gat_v)
                vals = plsc.load_gather(
                    gat_v, [lax.iota(jnp.int32, L), jnp.zeros((L,), jnp.int32)])
                out_v[...] = vals
                pltpu.sync_copy(out_v, out_hbm)

    return sc_topk


def kernel(a_out, v_out, seq_len, W, b):
    Bn, T, D = a_out.shape
    a_sls, v_sls, av_sls, keys = _tc_call(a_out, v_out, seq_len, W, b)
    mil_vls = _make_sc_topk(Bn, T)(keys.reshape(Bn, T), seq_len)
    return (mil_vls, a_sls, v_sls, av_sls)
